# baseline (speedup 1.0000x reference)
import functools

import jax
import jax.numpy as jnp
from jax import lax
from jax.experimental import pallas as pl
from jax.experimental.pallas import tpu as pltpu
from jax.experimental.pallas import tpu_sc as plsc

DEPTH = 12
UNITS = 32
N_NODES = 10000
N_EDGES = 320000

NC = 2
NS = 16
NW = NC * NS
EPW = N_EDGES // NW
G = 80
NCH = EPW // G
NRD = 10
ROWS_RD = N_NODES // NRD
CNT_W = 16


def _sigmoid(v):
    return 1.0 / (1.0 + jnp.exp(-v))


def _sc_edge_body(src_hbm, dst_hbm, s_hbm, x2_hbm, x3_hbm, x4_hbm,
                  g_hbm, aggp_hbm,
                  srcv, dstv, wv, r2, r3, r4, cbuf, gbuf, ob,
                  shared_agg, sem0, sem1, sem2):
    cid = lax.axis_index("c")
    sid = lax.axis_index("s")
    wid = sid * NC + cid

    @pl.when(sid < NRD)
    def _zero():
        def _zrow(r, _):
            z = jnp.zeros((16,), jnp.float32)
            ob[r, pl.ds(0, 16)] = z
            ob[r, pl.ds(16, 16)] = z
            return _
        lax.fori_loop(0, ROWS_RD, _zrow, None)
        pltpu.sync_copy(ob, shared_agg.at[pl.ds(sid * ROWS_RD, ROWS_RD)])

    plsc.subcore_barrier()

    base = wid * EPW

    def _group(gi, _):
        row0 = base + gi * G
        pltpu.sync_copy(src_hbm.at[pl.ds(row0, G)], srcv)
        pltpu.sync_copy(dst_hbm.at[pl.ds(row0, G)], dstv)
        c2 = pltpu.async_copy(x2_hbm.at[dstv], r2, sem0)
        c3 = pltpu.async_copy(x3_hbm.at[srcv], r3, sem1)
        c4 = pltpu.async_copy(x4_hbm.at[dstv], r4, sem2)
        pltpu.sync_copy(s_hbm.at[pl.ds(row0, G)], wv)
        c2.wait()
        c3.wait()
        c4.wait()

        def _row(r, _):
            for h in (0, 16):
                cbuf[r, pl.ds(h, 16)] = wv[r, pl.ds(h, 16)] * r2[r, pl.ds(h, 16)]
                gbuf[r, pl.ds(h, 16)] = r3[r, pl.ds(h, 16)] + r4[r, pl.ds(h, 16)]
            return _
        lax.fori_loop(0, G, _row, None)

        pltpu.sync_copy(gbuf, g_hbm.at[pl.ds(row0, G)])
        pltpu.sync_copy(cbuf, shared_agg.at[srcv], add=True)
        return _

    lax.fori_loop(0, NCH, _group, None)
    plsc.subcore_barrier()

    @pl.when(sid < NRD)
    def _readout():
        pltpu.sync_copy(shared_agg.at[pl.ds(sid * ROWS_RD, ROWS_RD)], ob)
        pltpu.sync_copy(ob, aggp_hbm.at[cid, pl.ds(sid * ROWS_RD, ROWS_RD)])


_sc_edge_kernel = functools.partial(
    pl.kernel,
    _sc_edge_body,
    out_type=[
        jax.ShapeDtypeStruct((N_EDGES, UNITS), jnp.float32),
        jax.ShapeDtypeStruct((NC, N_NODES, UNITS), jnp.float32),
    ],
    mesh=plsc.VectorSubcoreMesh(core_axis_name="c", subcore_axis_name="s"),
    compiler_params=pltpu.CompilerParams(use_tc_tiling_on_sc=False),
    scratch_types=[
        pltpu.VMEM((G,), jnp.int32),
        pltpu.VMEM((G,), jnp.int32),
        pltpu.VMEM((G, UNITS), jnp.float32),
        pltpu.VMEM((G, UNITS), jnp.float32),
        pltpu.VMEM((G, UNITS), jnp.float32),
        pltpu.VMEM((G, UNITS), jnp.float32),
        pltpu.VMEM((G, UNITS), jnp.float32),
        pltpu.VMEM((G, UNITS), jnp.float32),
        pltpu.VMEM((ROWS_RD, UNITS), jnp.float32),
        pltpu.VMEM_SHARED((N_NODES, UNITS), jnp.float32),
        pltpu.SemaphoreType.DMA,
        pltpu.SemaphoreType.DMA,
        pltpu.SemaphoreType.DMA,
    ],
)()


def _sc_cnt_body(src_hbm, cntp_hbm, srcv, ones_buf, ob, shared_cnt):
    cid = lax.axis_index("c")
    sid = lax.axis_index("s")
    wid = sid * NC + cid

    @pl.when(sid < NRD)
    def _zero():
        def _zrow(r, _):
            ob[r, pl.ds(0, 16)] = jnp.zeros((16,), jnp.float32)
            return _
        lax.fori_loop(0, ROWS_RD, _zrow, None)
        pltpu.sync_copy(ob, shared_cnt.at[pl.ds(sid * ROWS_RD, ROWS_RD)])

    def _fill1(r, _):
        ones_buf[r, pl.ds(0, 16)] = jnp.ones((16,), jnp.float32)
        return _
    lax.fori_loop(0, G, _fill1, None)
    plsc.subcore_barrier()

    base = wid * EPW

    def _group(gi, _):
        pltpu.sync_copy(src_hbm.at[pl.ds(base + gi * G, G)], srcv)
        pltpu.sync_copy(ones_buf, shared_cnt.at[srcv], add=True)
        return _

    lax.fori_loop(0, NCH, _group, None)
    plsc.subcore_barrier()

    @pl.when(sid < NRD)
    def _readout():
        pltpu.sync_copy(shared_cnt.at[pl.ds(sid * ROWS_RD, ROWS_RD)], ob)
        pltpu.sync_copy(ob, cntp_hbm.at[cid, pl.ds(sid * ROWS_RD, ROWS_RD)])


_sc_cnt_kernel = functools.partial(
    pl.kernel,
    _sc_cnt_body,
    out_type=[jax.ShapeDtypeStruct((NC, N_NODES, CNT_W), jnp.float32)],
    mesh=plsc.VectorSubcoreMesh(core_axis_name="c", subcore_axis_name="s"),
    compiler_params=pltpu.CompilerParams(use_tc_tiling_on_sc=False),
    scratch_types=[
        pltpu.VMEM((G,), jnp.int32),
        pltpu.VMEM((G, CNT_W), jnp.float32),
        pltpu.VMEM((ROWS_RD, CNT_W), jnp.float32),
        pltpu.VMEM_SHARED((N_NODES, CNT_W), jnp.float32),
    ],
)()


def _se(h, W1, W2):
    y = jax.nn.sigmoid(jax.nn.relu(h @ W1.T) @ W2.T)
    return h * y


def _bn(h, g, b, eps=1e-5):
    mu = jnp.mean(h, axis=0)
    var = jnp.var(h, axis=0)
    return g * (h - mu) * jax.lax.rsqrt(var + eps) + b


def kernel(x, edge_index, edge_attr, v_lin0_W, v_lin0_b, v1_W, v1_b, v2_W, v2_b, v3_W, v3_b, v4_W, v4_b, vbn_g, vbn_b, e_lin0_W, e_lin0_b, e0_W, e0_b, ebn_g, ebn_b, se_W1, se_W2):
    src = edge_index[0]
    dst = edge_index[1]

    cntp = _sc_cnt_kernel(src)[0]
    cnt = jnp.maximum(cntp[0, :, 0] + cntp[1, :, 0], 1.0)[:, None]

    h = jax.nn.silu(x @ v_lin0_W.T + v_lin0_b)
    w = jax.nn.silu(edge_attr @ e_lin0_W.T + e_lin0_b)

    for i in range(DEPTH):
        x0 = h
        x1 = _se(x0 @ v1_W[i].T + v1_b[i], se_W1, se_W2)
        x2 = _se(x0 @ v2_W[i].T + v2_b[i], se_W1, se_W2)
        x3 = _se(x0 @ v3_W[i].T + v3_b[i], se_W1, se_W2)
        x4 = _se(x0 @ v4_W[i].T + v4_b[i], se_W1, se_W2)
        w0 = w
        s = jax.nn.sigmoid(w0)

        g, aggp = _sc_edge_kernel(src, dst, s, x2, x3, x4)
        agg = (aggp[0] + aggp[1]) / cnt

        w1 = w0 @ e0_W[i].T + e0_b[i]
        h = x0 + jax.nn.silu(_bn(x1 + agg, vbn_g[i], vbn_b[i]))
        w = w0 + jax.nn.silu(_bn(w1 + g, ebn_g[i], ebn_b[i]))
    return w


# speedup vs baseline: 1.9324x; 1.9324x over previous
import functools

import jax
import jax.numpy as jnp
from jax import lax
from jax.experimental import pallas as pl
from jax.experimental.pallas import tpu as pltpu
from jax.experimental.pallas import tpu_sc as plsc

DEPTH = 12
UNITS = 32
N_NODES = 10000
N_EDGES = 320000

NC = 2
NS = 16
NW = NC * NS
EPW = N_EDGES // NW
G = 80
NCH = EPW // G
NRD = 10
ROWS_RD = N_NODES // NRD
CNT_W = 16


def _sigmoid(v):
    return 1.0 / (1.0 + jnp.exp(-v))


def _sc_edge_body(src_hbm, dst_hbm, s_hbm, x2_hbm, x3_hbm, x4_hbm,
                  g_hbm, aggp_hbm,
                  srcv, dstv, wv, r2, r3, r4, cbuf, gbuf, ob,
                  shared_agg, sem0, sem1, sem2):
    cid = lax.axis_index("c")
    sid = lax.axis_index("s")
    wid = sid * NC + cid

    @pl.when(sid < NRD)
    def _zero():
        def _zrow(r, _):
            z = jnp.zeros((16,), jnp.float32)
            ob[r, pl.ds(0, 16)] = z
            ob[r, pl.ds(16, 16)] = z
            return _
        lax.fori_loop(0, ROWS_RD, _zrow, None)
        pltpu.sync_copy(ob, shared_agg.at[pl.ds(sid * ROWS_RD, ROWS_RD)])

    plsc.subcore_barrier()

    base = wid * EPW

    def _group(gi, _):
        row0 = base + gi * G
        pltpu.sync_copy(src_hbm.at[pl.ds(row0, G)], srcv)
        pltpu.sync_copy(dst_hbm.at[pl.ds(row0, G)], dstv)
        c2 = pltpu.async_copy(x2_hbm.at[dstv], r2, sem0)
        c3 = pltpu.async_copy(x3_hbm.at[srcv], r3, sem1)
        c4 = pltpu.async_copy(x4_hbm.at[dstv], r4, sem2)
        pltpu.sync_copy(s_hbm.at[pl.ds(row0, G)], wv)
        c2.wait()
        c3.wait()
        c4.wait()

        def _row(r, _):
            for h in (0, 16):
                cbuf[r, pl.ds(h, 16)] = wv[r, pl.ds(h, 16)] * r2[r, pl.ds(h, 16)]
                gbuf[r, pl.ds(h, 16)] = r3[r, pl.ds(h, 16)] + r4[r, pl.ds(h, 16)]
            return _
        lax.fori_loop(0, G, _row, None)

        pltpu.sync_copy(gbuf, g_hbm.at[pl.ds(row0, G)])
        pltpu.sync_copy(cbuf, shared_agg.at[srcv], add=True)
        return _

    lax.fori_loop(0, NCH, _group, None)
    plsc.subcore_barrier()

    @pl.when(sid < NRD)
    def _readout():
        pltpu.sync_copy(shared_agg.at[pl.ds(sid * ROWS_RD, ROWS_RD)], ob)
        pltpu.sync_copy(ob, aggp_hbm.at[cid, pl.ds(sid * ROWS_RD, ROWS_RD)])


_sc_edge_kernel = functools.partial(
    pl.kernel,
    _sc_edge_body,
    out_type=[
        jax.ShapeDtypeStruct((N_EDGES, UNITS), jnp.float32),
        jax.ShapeDtypeStruct((NC, N_NODES, UNITS), jnp.float32),
    ],
    mesh=plsc.VectorSubcoreMesh(core_axis_name="c", subcore_axis_name="s"),
    compiler_params=pltpu.CompilerParams(use_tc_tiling_on_sc=False),
    scratch_types=[
        pltpu.VMEM((G,), jnp.int32),
        pltpu.VMEM((G,), jnp.int32),
        pltpu.VMEM((G, UNITS), jnp.float32),
        pltpu.VMEM((G, UNITS), jnp.float32),
        pltpu.VMEM((G, UNITS), jnp.float32),
        pltpu.VMEM((G, UNITS), jnp.float32),
        pltpu.VMEM((G, UNITS), jnp.float32),
        pltpu.VMEM((G, UNITS), jnp.float32),
        pltpu.VMEM((ROWS_RD, UNITS), jnp.float32),
        pltpu.VMEM_SHARED((N_NODES, UNITS), jnp.float32),
        pltpu.SemaphoreType.DMA,
        pltpu.SemaphoreType.DMA,
        pltpu.SemaphoreType.DMA,
    ],
)()


def _sc_cnt_body(src_hbm, cntp_hbm, srcv, ones_buf, ob, shared_cnt):
    cid = lax.axis_index("c")
    sid = lax.axis_index("s")
    wid = sid * NC + cid

    @pl.when(sid < NRD)
    def _zero():
        def _zrow(r, _):
            ob[r, pl.ds(0, 16)] = jnp.zeros((16,), jnp.float32)
            return _
        lax.fori_loop(0, ROWS_RD, _zrow, None)
        pltpu.sync_copy(ob, shared_cnt.at[pl.ds(sid * ROWS_RD, ROWS_RD)])

    def _fill1(r, _):
        ones_buf[r, pl.ds(0, 16)] = jnp.ones((16,), jnp.float32)
        return _
    lax.fori_loop(0, G, _fill1, None)
    plsc.subcore_barrier()

    base = wid * EPW

    def _group(gi, _):
        pltpu.sync_copy(src_hbm.at[pl.ds(base + gi * G, G)], srcv)
        pltpu.sync_copy(ones_buf, shared_cnt.at[srcv], add=True)
        return _

    lax.fori_loop(0, NCH, _group, None)
    plsc.subcore_barrier()

    @pl.when(sid < NRD)
    def _readout():
        pltpu.sync_copy(shared_cnt.at[pl.ds(sid * ROWS_RD, ROWS_RD)], ob)
        pltpu.sync_copy(ob, cntp_hbm.at[cid, pl.ds(sid * ROWS_RD, ROWS_RD)])


_sc_cnt_kernel = functools.partial(
    pl.kernel,
    _sc_cnt_body,
    out_type=[jax.ShapeDtypeStruct((NC, N_NODES, CNT_W), jnp.float32)],
    mesh=plsc.VectorSubcoreMesh(core_axis_name="c", subcore_axis_name="s"),
    compiler_params=pltpu.CompilerParams(use_tc_tiling_on_sc=False),
    scratch_types=[
        pltpu.VMEM((G,), jnp.int32),
        pltpu.VMEM((G, CNT_W), jnp.float32),
        pltpu.VMEM((ROWS_RD, CNT_W), jnp.float32),
        pltpu.VMEM_SHARED((N_NODES, CNT_W), jnp.float32),
    ],
)()


def _se(h, W1, W2):
    y = jax.nn.sigmoid(jax.nn.relu(h @ W1.T) @ W2.T)
    return h * y


def _bn(h, g, b, eps=1e-5):
    mu = jnp.mean(h, axis=0)
    var = jnp.var(h, axis=0)
    return g * (h - mu) * jax.lax.rsqrt(var + eps) + b


EV = N_EDGES // 4
ECH = 2000
NEC = EV // ECH


def _tc_edge_body(w0_ref, g_ref, Wblk_ref, b_ref, bng_ref, bnb_ref, F_ref,
                  wn_ref, sn_ref, t_all, sum_ref, sq_ref):
    p = pl.program_id(0)
    k = pl.program_id(1)

    @pl.when(p == 0)
    def _pass1():
        @pl.when(k == 0)
        def _init():
            sum_ref[...] = jnp.zeros_like(sum_ref)
            sq_ref[...] = jnp.zeros_like(sq_ref)

        t = (jnp.dot(w0_ref[...], Wblk_ref[...],
                     preferred_element_type=jnp.float32)
             + b_ref[...] + g_ref[...])
        t_all[pl.ds(k * ECH, ECH), :] = t
        sum_ref[...] += jnp.sum(t, axis=0, keepdims=True)
        sq_ref[...] += jnp.sum(t * t, axis=0, keepdims=True)

    @pl.when(p == 1)
    def _pass2():
        inv_n = 1.0 / N_EDGES
        mu = jnp.dot(sum_ref[...], F_ref[...],
                     preferred_element_type=jnp.float32) * inv_n
        ex2 = jnp.dot(sq_ref[...], F_ref[...],
                      preferred_element_type=jnp.float32) * inv_n
        inv = lax.rsqrt(ex2 - mu * mu + 1e-5)
        t = t_all[pl.ds(k * ECH, ECH), :]
        y = bng_ref[...] * (t - mu) * inv + bnb_ref[...]
        o = w0_ref[...] + y * (1.0 / (1.0 + jnp.exp(-y)))
        wn_ref[...] = o
        sn_ref[...] = 1.0 / (1.0 + jnp.exp(-o))


def _tc_edge_call(w0v, gv, Wblk, b128, bng128, bnb128, F):
    row = pl.BlockSpec((ECH, 128), lambda p, k: (k, 0))
    row_p0 = pl.BlockSpec((ECH, 128), lambda p, k: (k * (1 - p), 0))
    row_p1 = pl.BlockSpec((ECH, 128), lambda p, k: (k * p, 0))
    vec = pl.BlockSpec((1, 128), lambda p, k: (0, 0))
    mat = pl.BlockSpec((128, 128), lambda p, k: (0, 0))
    return pl.pallas_call(
        _tc_edge_body,
        grid=(2, NEC),
        in_specs=[row, row_p0, mat, vec, vec, vec, mat],
        out_specs=[row_p1, row_p1],
        out_shape=[jax.ShapeDtypeStruct((EV, 128), jnp.float32),
                   jax.ShapeDtypeStruct((EV, 128), jnp.float32)],
        scratch_shapes=[pltpu.VMEM((EV, 128), jnp.float32),
                        pltpu.VMEM((1, 128), jnp.float32),
                        pltpu.VMEM((1, 128), jnp.float32)],
    )(w0v, gv, Wblk, b128, bng128, bnb128, F)


def kernel(x, edge_index, edge_attr, v_lin0_W, v_lin0_b, v1_W, v1_b, v2_W, v2_b, v3_W, v3_b, v4_W, v4_b, vbn_g, vbn_b, e_lin0_W, e_lin0_b, e0_W, e0_b, ebn_g, ebn_b, se_W1, se_W2):
    src = edge_index[0]
    dst = edge_index[1]

    cntp = _sc_cnt_kernel(src)[0]
    cnt = jnp.maximum(cntp[0, :, 0] + cntp[1, :, 0], 1.0)[:, None]

    h = jax.nn.silu(x @ v_lin0_W.T + v_lin0_b)
    w = jax.nn.silu(edge_attr @ e_lin0_W.T + e_lin0_b)
    s = jax.nn.sigmoid(w)
    wv = w.reshape(EV, 128)

    eye4 = jnp.eye(4, dtype=jnp.float32)
    F = jnp.tile(jnp.eye(32, dtype=jnp.float32), (4, 4))
    Wblk = jnp.einsum("ab,iuv->iaubv", eye4, jnp.transpose(e0_W, (0, 2, 1)))
    Wblk = Wblk.reshape(DEPTH, 128, 128)
    eb128 = jnp.tile(e0_b, (1, 4)).reshape(DEPTH, 1, 128)
    ebg128 = jnp.tile(ebn_g, (1, 4)).reshape(DEPTH, 1, 128)
    ebb128 = jnp.tile(ebn_b, (1, 4)).reshape(DEPTH, 1, 128)

    for i in range(DEPTH):
        x0 = h
        x1 = _se(x0 @ v1_W[i].T + v1_b[i], se_W1, se_W2)
        x2 = _se(x0 @ v2_W[i].T + v2_b[i], se_W1, se_W2)
        x3 = _se(x0 @ v3_W[i].T + v3_b[i], se_W1, se_W2)
        x4 = _se(x0 @ v4_W[i].T + v4_b[i], se_W1, se_W2)

        g, aggp = _sc_edge_kernel(src, dst, s, x2, x3, x4)
        agg = (aggp[0] + aggp[1]) / cnt

        wv, sv = _tc_edge_call(wv, g.reshape(EV, 128), Wblk[i], eb128[i],
                               ebg128[i], ebb128[i], F)
        s = sv.reshape(N_EDGES, UNITS)
        h = x0 + jax.nn.silu(_bn(x1 + agg, vbn_g[i], vbn_b[i]))
    return wv.reshape(N_EDGES, UNITS)


# speedup vs baseline: 3.3452x; 1.7311x over previous
import functools

import jax
import jax.numpy as jnp
from jax import lax
from jax.experimental import pallas as pl
from jax.experimental.pallas import tpu as pltpu
from jax.experimental.pallas import tpu_sc as plsc

DEPTH = 12
UNITS = 32
N_NODES = 10000
N_EDGES = 320000

NC = 2
NS = 16
NW = NC * NS
EPW = N_EDGES // NW
G = 80
NCH = EPW // G
NBUF = 5
NRD = 10
ROWS_RD = N_NODES // NRD
RD2 = ROWS_RD // 2
CNT_W = 16


def _sigmoid(v):
    return 1.0 / (1.0 + jnp.exp(-v))


def _sc_edge_body(src3_hbm, dst3_hbm, s_hbm, x2_hbm, x3_hbm, x4_hbm,
                  g_hbm, aggp_hbm,
                  srcA, dstA, wv, r2, r3, r4, cbuf, gbuf, ob,
                  shared_agg, lsem, wsem):
    cid = lax.axis_index("c")
    sid = lax.axis_index("s")
    wid = sid * NC + cid
    base = wid * EPW

    @pl.when(sid < NRD)
    def _zero():
        def _zrow(r, _):
            z = jnp.zeros((16,), jnp.float32)
            ob[r, pl.ds(0, 16)] = z
            ob[r, pl.ds(16, 16)] = z
            return _
        lax.fori_loop(0, RD2, _zrow, None)
        pltpu.sync_copy(ob, shared_agg.at[pl.ds(sid * ROWS_RD, RD2)])
        pltpu.sync_copy(ob, shared_agg.at[pl.ds(sid * ROWS_RD + RD2, RD2)])

    pltpu.sync_copy(src3_hbm.at[wid], srcA)
    pltpu.sync_copy(dst3_hbm.at[wid], dstA)
    plsc.subcore_barrier()

    def _fire(gi, b):
        pltpu.async_copy(x2_hbm.at[dstA.at[gi]], r2.at[b], lsem.at[b])
        pltpu.async_copy(x3_hbm.at[srcA.at[gi]], r3.at[b], lsem.at[b])
        pltpu.async_copy(x4_hbm.at[dstA.at[gi]], r4.at[b], lsem.at[b])
        pltpu.async_copy(s_hbm.at[pl.ds(base + gi * G, G)], wv.at[b], lsem.at[b])

    def _drain_loads(b):
        for _ in range(4):
            pltpu.make_async_copy(
                s_hbm.at[pl.ds(base, G)], wv.at[b], lsem.at[b]).wait()

    def _drain_gwrite(b):
        pltpu.make_async_copy(
            gbuf.at[b], g_hbm.at[pl.ds(base, G)], wsem.at[b]).wait()

    for b in range(NBUF):
        _fire(b, b)

    def _outer(it, _):
        g0 = it * NBUF
        for b in range(NBUF):
            gi = g0 + b
            _drain_loads(b)

            @pl.when(it > 0)
            def _w():
                _drain_gwrite(b)

            def _row(r, _):
                for h in (0, 16):
                    cbuf[r, pl.ds(h, 16)] = (
                        wv[b, r, pl.ds(h, 16)] * r2[b, r, pl.ds(h, 16)])
                    gbuf[b, r, pl.ds(h, 16)] = (
                        r3[b, r, pl.ds(h, 16)] + r4[b, r, pl.ds(h, 16)])
                return _
            lax.fori_loop(0, G, _row, None)

            @pl.when(gi + NBUF < NCH)
            def _f():
                _fire(gi + NBUF, b)

            pltpu.async_copy(gbuf.at[b], g_hbm.at[pl.ds(base + gi * G, G)],
                             wsem.at[b])
            pltpu.sync_copy(cbuf, shared_agg.at[srcA.at[gi]], add=True)
        return _

    lax.fori_loop(0, NCH // NBUF, _outer, None)
    for b in range(NBUF):
        _drain_gwrite(b)
    plsc.subcore_barrier()

    @pl.when(sid < NRD)
    def _readout():
        for half in range(2):
            pltpu.sync_copy(
                shared_agg.at[pl.ds(sid * ROWS_RD + half * RD2, RD2)], ob)
            pltpu.sync_copy(
                ob, aggp_hbm.at[cid, pl.ds(sid * ROWS_RD + half * RD2, RD2)])


_sc_edge_kernel = functools.partial(
    pl.kernel,
    _sc_edge_body,
    out_type=[
        jax.ShapeDtypeStruct((N_EDGES, UNITS), jnp.float32),
        jax.ShapeDtypeStruct((NC, N_NODES, UNITS), jnp.float32),
    ],
    mesh=plsc.VectorSubcoreMesh(core_axis_name="c", subcore_axis_name="s"),
    compiler_params=pltpu.CompilerParams(use_tc_tiling_on_sc=False),
    scratch_types=[
        pltpu.VMEM((NCH, G), jnp.int32),
        pltpu.VMEM((NCH, G), jnp.int32),
        pltpu.VMEM((NBUF, G, UNITS), jnp.float32),
        pltpu.VMEM((NBUF, G, UNITS), jnp.float32),
        pltpu.VMEM((NBUF, G, UNITS), jnp.float32),
        pltpu.VMEM((NBUF, G, UNITS), jnp.float32),
        pltpu.VMEM((G, UNITS), jnp.float32),
        pltpu.VMEM((NBUF, G, UNITS), jnp.float32),
        pltpu.VMEM((RD2, UNITS), jnp.float32),
        pltpu.VMEM_SHARED((N_NODES, UNITS), jnp.float32),
        pltpu.SemaphoreType.DMA((NBUF,)),
        pltpu.SemaphoreType.DMA((NBUF,)),
    ],
)()


def _sc_cnt_body(src_hbm, cntp_hbm, srcv, ones_buf, ob, shared_cnt):
    cid = lax.axis_index("c")
    sid = lax.axis_index("s")
    wid = sid * NC + cid

    @pl.when(sid < NRD)
    def _zero():
        def _zrow(r, _):
            ob[r, pl.ds(0, 16)] = jnp.zeros((16,), jnp.float32)
            return _
        lax.fori_loop(0, ROWS_RD, _zrow, None)
        pltpu.sync_copy(ob, shared_cnt.at[pl.ds(sid * ROWS_RD, ROWS_RD)])

    def _fill1(r, _):
        ones_buf[r, pl.ds(0, 16)] = jnp.ones((16,), jnp.float32)
        return _
    lax.fori_loop(0, G, _fill1, None)
    plsc.subcore_barrier()

    base = wid * EPW

    def _group(gi, _):
        pltpu.sync_copy(src_hbm.at[pl.ds(base + gi * G, G)], srcv)
        pltpu.sync_copy(ones_buf, shared_cnt.at[srcv], add=True)
        return _

    lax.fori_loop(0, NCH, _group, None)
    plsc.subcore_barrier()

    @pl.when(sid < NRD)
    def _readout():
        pltpu.sync_copy(shared_cnt.at[pl.ds(sid * ROWS_RD, ROWS_RD)], ob)
        pltpu.sync_copy(ob, cntp_hbm.at[cid, pl.ds(sid * ROWS_RD, ROWS_RD)])


_sc_cnt_kernel = functools.partial(
    pl.kernel,
    _sc_cnt_body,
    out_type=[jax.ShapeDtypeStruct((NC, N_NODES, CNT_W), jnp.float32)],
    mesh=plsc.VectorSubcoreMesh(core_axis_name="c", subcore_axis_name="s"),
    compiler_params=pltpu.CompilerParams(use_tc_tiling_on_sc=False),
    scratch_types=[
        pltpu.VMEM((G,), jnp.int32),
        pltpu.VMEM((G, CNT_W), jnp.float32),
        pltpu.VMEM((ROWS_RD, CNT_W), jnp.float32),
        pltpu.VMEM_SHARED((N_NODES, CNT_W), jnp.float32),
    ],
)()


def _se(h, W1, W2):
    y = jax.nn.sigmoid(jax.nn.relu(h @ W1.T) @ W2.T)
    return h * y


def _bn(h, g, b, eps=1e-5):
    mu = jnp.mean(h, axis=0)
    var = jnp.var(h, axis=0)
    return g * (h - mu) * jax.lax.rsqrt(var + eps) + b


EV = N_EDGES // 4
ECH = 2000
NEC = EV // ECH


def _tc_edge_body(w0_ref, g_ref, Wblk_ref, b_ref, bng_ref, bnb_ref, F_ref,
                  wn_ref, sn_ref, t_all, sum_ref, sq_ref):
    p = pl.program_id(0)
    k = pl.program_id(1)

    @pl.when(p == 0)
    def _pass1():
        @pl.when(k == 0)
        def _init():
            sum_ref[...] = jnp.zeros_like(sum_ref)
            sq_ref[...] = jnp.zeros_like(sq_ref)

        t = (jnp.dot(w0_ref[...], Wblk_ref[...],
                     preferred_element_type=jnp.float32)
             + b_ref[...] + g_ref[...])
        t_all[pl.ds(k * ECH, ECH), :] = t
        sum_ref[...] += jnp.sum(t, axis=0, keepdims=True)
        sq_ref[...] += jnp.sum(t * t, axis=0, keepdims=True)

    @pl.when(p == 1)
    def _pass2():
        inv_n = 1.0 / N_EDGES
        mu = jnp.dot(sum_ref[...], F_ref[...],
                     preferred_element_type=jnp.float32) * inv_n
        ex2 = jnp.dot(sq_ref[...], F_ref[...],
                      preferred_element_type=jnp.float32) * inv_n
        inv = lax.rsqrt(ex2 - mu * mu + 1e-5)
        t = t_all[pl.ds(k * ECH, ECH), :]
        y = bng_ref[...] * (t - mu) * inv + bnb_ref[...]
        o = w0_ref[...] + y * (1.0 / (1.0 + jnp.exp(-y)))
        wn_ref[...] = o
        sn_ref[...] = 1.0 / (1.0 + jnp.exp(-o))


def _tc_edge_call(w0v, gv, Wblk, b128, bng128, bnb128, F):
    row = pl.BlockSpec((ECH, 128), lambda p, k: (k, 0))
    row_p0 = pl.BlockSpec((ECH, 128), lambda p, k: (k * (1 - p), 0))
    row_p1 = pl.BlockSpec((ECH, 128), lambda p, k: (k * p, 0))
    vec = pl.BlockSpec((1, 128), lambda p, k: (0, 0))
    mat = pl.BlockSpec((128, 128), lambda p, k: (0, 0))
    return pl.pallas_call(
        _tc_edge_body,
        grid=(2, NEC),
        in_specs=[row, row_p0, mat, vec, vec, vec, mat],
        out_specs=[row_p1, row_p1],
        out_shape=[jax.ShapeDtypeStruct((EV, 128), jnp.float32),
                   jax.ShapeDtypeStruct((EV, 128), jnp.float32)],
        scratch_shapes=[pltpu.VMEM((EV, 128), jnp.float32),
                        pltpu.VMEM((1, 128), jnp.float32),
                        pltpu.VMEM((1, 128), jnp.float32)],
    )(w0v, gv, Wblk, b128, bng128, bnb128, F)


def kernel(x, edge_index, edge_attr, v_lin0_W, v_lin0_b, v1_W, v1_b, v2_W, v2_b, v3_W, v3_b, v4_W, v4_b, vbn_g, vbn_b, e_lin0_W, e_lin0_b, e0_W, e0_b, ebn_g, ebn_b, se_W1, se_W2):
    src = edge_index[0]
    dst = edge_index[1]
    src3 = src.reshape(NW, NCH, G)
    dst3 = dst.reshape(NW, NCH, G)

    cntp = _sc_cnt_kernel(src)[0]
    cnt = jnp.maximum(cntp[0, :, 0] + cntp[1, :, 0], 1.0)[:, None]

    h = jax.nn.silu(x @ v_lin0_W.T + v_lin0_b)
    w = jax.nn.silu(edge_attr @ e_lin0_W.T + e_lin0_b)
    s = jax.nn.sigmoid(w)
    wv = w.reshape(EV, 128)

    eye4 = jnp.eye(4, dtype=jnp.float32)
    F = jnp.tile(jnp.eye(32, dtype=jnp.float32), (4, 4))
    Wblk = jnp.einsum("ab,iuv->iaubv", eye4, jnp.transpose(e0_W, (0, 2, 1)))
    Wblk = Wblk.reshape(DEPTH, 128, 128)
    eb128 = jnp.tile(e0_b, (1, 4)).reshape(DEPTH, 1, 128)
    ebg128 = jnp.tile(ebn_g, (1, 4)).reshape(DEPTH, 1, 128)
    ebb128 = jnp.tile(ebn_b, (1, 4)).reshape(DEPTH, 1, 128)

    for i in range(DEPTH):
        x0 = h
        x1 = _se(x0 @ v1_W[i].T + v1_b[i], se_W1, se_W2)
        x2 = _se(x0 @ v2_W[i].T + v2_b[i], se_W1, se_W2)
        x3 = _se(x0 @ v3_W[i].T + v3_b[i], se_W1, se_W2)
        x4 = _se(x0 @ v4_W[i].T + v4_b[i], se_W1, se_W2)

        g, aggp = _sc_edge_kernel(src3, dst3, s, x2, x3, x4)
        agg = (aggp[0] + aggp[1]) / cnt

        wv, sv = _tc_edge_call(wv, g.reshape(EV, 128), Wblk[i], eb128[i],
                               ebg128[i], ebb128[i], F)
        s = sv.reshape(N_EDGES, UNITS)
        h = x0 + jax.nn.silu(_bn(x1 + agg, vbn_g[i], vbn_b[i]))
    return wv.reshape(N_EDGES, UNITS)


# speedup vs baseline: 3.6477x; 1.0904x over previous
import functools

import jax
import jax.numpy as jnp
from jax import lax
from jax.experimental import pallas as pl
from jax.experimental.pallas import tpu as pltpu
from jax.experimental.pallas import tpu_sc as plsc

DEPTH = 12
UNITS = 32
N_NODES = 10000
N_EDGES = 320000

NC = 2
NS = 16
NW = NC * NS
EPW = N_EDGES // NW
G = 80
NCH = EPW // G
NBUF = 5
NRD = 10
ROWS_RD = N_NODES // NRD
RD2 = ROWS_RD // 2
CNT_W = 16


def _sigmoid(v):
    return 1.0 / (1.0 + jnp.exp(-v))


def _sc_edge_body(src3_hbm, dst3_hbm, s_hbm, x2_hbm, x3_hbm, x4_hbm,
                  g_hbm, aggp_hbm,
                  srcA, dstA, wv, r2, r3, r4, cbuf, gbuf, ob,
                  shared_agg, lsem, wsem):
    cid = lax.axis_index("c")
    sid = lax.axis_index("s")
    wid = sid * NC + cid
    base = wid * EPW

    @pl.when(sid < NRD)
    def _zero():
        def _zrow(r, _):
            z = jnp.zeros((16,), jnp.float32)
            ob[r, pl.ds(0, 16)] = z
            ob[r, pl.ds(16, 16)] = z
            return _
        lax.fori_loop(0, RD2, _zrow, None)
        pltpu.sync_copy(ob, shared_agg.at[pl.ds(sid * ROWS_RD, RD2)])
        pltpu.sync_copy(ob, shared_agg.at[pl.ds(sid * ROWS_RD + RD2, RD2)])

    pltpu.sync_copy(src3_hbm.at[wid], srcA)
    pltpu.sync_copy(dst3_hbm.at[wid], dstA)
    plsc.subcore_barrier()

    def _fire(gi, b):
        pltpu.async_copy(x2_hbm.at[dstA.at[gi]], r2.at[b], lsem.at[b])
        pltpu.async_copy(x3_hbm.at[srcA.at[gi]], r3.at[b], lsem.at[b])
        pltpu.async_copy(x4_hbm.at[dstA.at[gi]], r4.at[b], lsem.at[b])
        pltpu.async_copy(s_hbm.at[pl.ds(base + gi * G, G)], wv.at[b], lsem.at[b])

    def _drain_loads(b):
        for _ in range(4):
            pltpu.make_async_copy(
                s_hbm.at[pl.ds(base, G)], wv.at[b], lsem.at[b]).wait()

    def _drain_gwrite(b):
        pltpu.make_async_copy(
            gbuf.at[b], g_hbm.at[pl.ds(base, G)], wsem.at[b]).wait()

    for b in range(NBUF):
        _fire(b, b)

    def _outer(it, _):
        g0 = it * NBUF
        for b in range(NBUF):
            gi = g0 + b
            _drain_loads(b)

            @pl.when(it > 0)
            def _w():
                _drain_gwrite(b)

            def _row(r, _):
                for h in (0, 16):
                    cbuf[r, pl.ds(h, 16)] = (
                        wv[b, r, pl.ds(h, 16)] * r2[b, r, pl.ds(h, 16)])
                    gbuf[b, r, pl.ds(h, 16)] = (
                        r3[b, r, pl.ds(h, 16)] + r4[b, r, pl.ds(h, 16)])
                return _
            lax.fori_loop(0, G, _row, None)

            @pl.when(gi + NBUF < NCH)
            def _f():
                _fire(gi + NBUF, b)

            pltpu.async_copy(gbuf.at[b], g_hbm.at[pl.ds(base + gi * G, G)],
                             wsem.at[b])
            pltpu.sync_copy(cbuf, shared_agg.at[srcA.at[gi]], add=True)
        return _

    lax.fori_loop(0, NCH // NBUF, _outer, None)
    for b in range(NBUF):
        _drain_gwrite(b)
    plsc.subcore_barrier()

    @pl.when(sid < NRD)
    def _readout():
        for half in range(2):
            pltpu.sync_copy(
                shared_agg.at[pl.ds(sid * ROWS_RD + half * RD2, RD2)], ob)
            pltpu.sync_copy(
                ob, aggp_hbm.at[cid, pl.ds(sid * ROWS_RD + half * RD2, RD2)])


_sc_edge_kernel = functools.partial(
    pl.kernel,
    _sc_edge_body,
    out_type=[
        jax.ShapeDtypeStruct((N_EDGES, UNITS), jnp.float32),
        jax.ShapeDtypeStruct((NC, N_NODES, UNITS), jnp.float32),
    ],
    mesh=plsc.VectorSubcoreMesh(core_axis_name="c", subcore_axis_name="s"),
    compiler_params=pltpu.CompilerParams(use_tc_tiling_on_sc=False),
    scratch_types=[
        pltpu.VMEM((NCH, G), jnp.int32),
        pltpu.VMEM((NCH, G), jnp.int32),
        pltpu.VMEM((NBUF, G, UNITS), jnp.float32),
        pltpu.VMEM((NBUF, G, UNITS), jnp.float32),
        pltpu.VMEM((NBUF, G, UNITS), jnp.float32),
        pltpu.VMEM((NBUF, G, UNITS), jnp.float32),
        pltpu.VMEM((G, UNITS), jnp.float32),
        pltpu.VMEM((NBUF, G, UNITS), jnp.float32),
        pltpu.VMEM((RD2, UNITS), jnp.float32),
        pltpu.VMEM_SHARED((N_NODES, UNITS), jnp.float32),
        pltpu.SemaphoreType.DMA((NBUF,)),
        pltpu.SemaphoreType.DMA((NBUF,)),
    ],
)()


def _sc_cnt_body(src_hbm, cntp_hbm, srcv, ones_buf, ob, shared_cnt):
    cid = lax.axis_index("c")
    sid = lax.axis_index("s")
    wid = sid * NC + cid

    @pl.when(sid < NRD)
    def _zero():
        def _zrow(r, _):
            ob[r, pl.ds(0, 16)] = jnp.zeros((16,), jnp.float32)
            return _
        lax.fori_loop(0, ROWS_RD, _zrow, None)
        pltpu.sync_copy(ob, shared_cnt.at[pl.ds(sid * ROWS_RD, ROWS_RD)])

    def _fill1(r, _):
        ones_buf[r, pl.ds(0, 16)] = jnp.ones((16,), jnp.float32)
        return _
    lax.fori_loop(0, G, _fill1, None)
    plsc.subcore_barrier()

    base = wid * EPW

    def _group(gi, _):
        pltpu.sync_copy(src_hbm.at[pl.ds(base + gi * G, G)], srcv)
        pltpu.sync_copy(ones_buf, shared_cnt.at[srcv], add=True)
        return _

    lax.fori_loop(0, NCH, _group, None)
    plsc.subcore_barrier()

    @pl.when(sid < NRD)
    def _readout():
        pltpu.sync_copy(shared_cnt.at[pl.ds(sid * ROWS_RD, ROWS_RD)], ob)
        pltpu.sync_copy(ob, cntp_hbm.at[cid, pl.ds(sid * ROWS_RD, ROWS_RD)])


_sc_cnt_kernel = functools.partial(
    pl.kernel,
    _sc_cnt_body,
    out_type=[jax.ShapeDtypeStruct((NC, N_NODES, CNT_W), jnp.float32)],
    mesh=plsc.VectorSubcoreMesh(core_axis_name="c", subcore_axis_name="s"),
    compiler_params=pltpu.CompilerParams(use_tc_tiling_on_sc=False),
    scratch_types=[
        pltpu.VMEM((G,), jnp.int32),
        pltpu.VMEM((G, CNT_W), jnp.float32),
        pltpu.VMEM((ROWS_RD, CNT_W), jnp.float32),
        pltpu.VMEM_SHARED((N_NODES, CNT_W), jnp.float32),
    ],
)()


NVB = 2000
NNB = N_NODES // NVB


def _node_mlp(hn, Wcat_ref, bcat_ref, W1blk_ref, W2blk_ref, S_refs,
              x_refs):
    xl = jnp.dot(hn, Wcat_ref[...], preferred_element_type=jnp.float32)
    xl = xl + bcat_ref[...]
    u = jnp.maximum(jnp.dot(xl, W1blk_ref[...],
                            preferred_element_type=jnp.float32), 0.0)
    y = 1.0 / (1.0 + jnp.exp(-jnp.dot(u, W2blk_ref[...],
                                      preferred_element_type=jnp.float32)))
    xall = xl * y
    for S_ref, x_ref in zip(S_refs, x_refs):
        x_ref[...] = jnp.dot(xall, S_ref[...],
                             preferred_element_type=jnp.float32)


def _tc_node_body(x0_ref, x1p_ref, aggp_ref, ci_ref, bng_ref, bnb_ref,
                  Wcat_ref, bcat_ref, W1blk_ref, W2blk_ref,
                  S1_ref, S2_ref, S3_ref, S4_ref,
                  h_ref, x1_ref, x2_ref, x3_ref, x4_ref,
                  z_all, sum_ref, sq_ref):
    p = pl.program_id(0)
    k = pl.program_id(1)

    @pl.when(p == 0)
    def _pass1():
        @pl.when(k == 0)
        def _init():
            sum_ref[...] = jnp.zeros_like(sum_ref)
            sq_ref[...] = jnp.zeros_like(sq_ref)

        z = x1p_ref[...] + (aggp_ref[0] + aggp_ref[1]) * ci_ref[...]
        z_all[pl.ds(k * NVB, NVB), :] = z
        sum_ref[...] += jnp.sum(z, axis=0, keepdims=True)
        sq_ref[...] += jnp.sum(z * z, axis=0, keepdims=True)

    @pl.when(p == 1)
    def _pass2():
        inv_n = 1.0 / N_NODES
        mu = sum_ref[...] * inv_n
        inv = lax.rsqrt(sq_ref[...] * inv_n - mu * mu + 1e-5)
        z = z_all[pl.ds(k * NVB, NVB), :]
        hb = bng_ref[...] * (z - mu) * inv + bnb_ref[...]
        hn = x0_ref[...] + hb * (1.0 / (1.0 + jnp.exp(-hb)))
        h_ref[...] = hn
        _node_mlp(hn, Wcat_ref, bcat_ref, W1blk_ref, W2blk_ref,
                  (S1_ref, S2_ref, S3_ref, S4_ref),
                  (x1_ref, x2_ref, x3_ref, x4_ref))


def _tc_node_call(x0, x1p, aggp, cntinv, bng, bnb, Wcat, bcat,
                  W1blk, W2blk, S1, S2, S3, S4):
    row_p0 = pl.BlockSpec((NVB, UNITS), lambda p, k: (k * (1 - p), 0))
    row_p1 = pl.BlockSpec((NVB, UNITS), lambda p, k: (k * p, 0))
    agg_sp = pl.BlockSpec((NC, NVB, UNITS), lambda p, k: (0, k * (1 - p), 0))
    ci_sp = pl.BlockSpec((NVB, 1), lambda p, k: (k * (1 - p), 0))
    vec = pl.BlockSpec((1, UNITS), lambda p, k: (0, 0))
    w_sp = pl.BlockSpec((UNITS, 128), lambda p, k: (0, 0))
    v128 = pl.BlockSpec((1, 128), lambda p, k: (0, 0))
    m128 = pl.BlockSpec((128, 128), lambda p, k: (0, 0))
    sel = pl.BlockSpec((128, UNITS), lambda p, k: (0, 0))
    out = jax.ShapeDtypeStruct((N_NODES, UNITS), jnp.float32)
    return pl.pallas_call(
        _tc_node_body,
        grid=(2, NNB),
        in_specs=[row_p1, row_p0, agg_sp, ci_sp, vec, vec,
                  w_sp, v128, m128, m128, sel, sel, sel, sel],
        out_specs=[row_p1] * 5,
        out_shape=[out] * 5,
        scratch_shapes=[pltpu.VMEM((N_NODES, UNITS), jnp.float32),
                        pltpu.VMEM((1, UNITS), jnp.float32),
                        pltpu.VMEM((1, UNITS), jnp.float32)],
    )(x0, x1p, aggp, cntinv, bng, bnb, Wcat, bcat, W1blk, W2blk,
      S1, S2, S3, S4)


def _tc_node0_body(h_ref, Wcat_ref, bcat_ref, W1blk_ref, W2blk_ref,
                   S1_ref, S2_ref, S3_ref, S4_ref,
                   x1_ref, x2_ref, x3_ref, x4_ref):
    _node_mlp(h_ref[...], Wcat_ref, bcat_ref, W1blk_ref, W2blk_ref,
              (S1_ref, S2_ref, S3_ref, S4_ref),
              (x1_ref, x2_ref, x3_ref, x4_ref))


def _tc_node0_call(h, Wcat, bcat, W1blk, W2blk, S1, S2, S3, S4):
    row = pl.BlockSpec((NVB, UNITS), lambda k: (k, 0))
    w_sp = pl.BlockSpec((UNITS, 128), lambda k: (0, 0))
    v128 = pl.BlockSpec((1, 128), lambda k: (0, 0))
    m128 = pl.BlockSpec((128, 128), lambda k: (0, 0))
    sel = pl.BlockSpec((128, UNITS), lambda k: (0, 0))
    out = jax.ShapeDtypeStruct((N_NODES, UNITS), jnp.float32)
    return pl.pallas_call(
        _tc_node0_body,
        grid=(NNB,),
        in_specs=[row, w_sp, v128, m128, m128, sel, sel, sel, sel],
        out_specs=[row] * 4,
        out_shape=[out] * 4,
    )(h, Wcat, bcat, W1blk, W2blk, S1, S2, S3, S4)


EV = N_EDGES // 4
ECH = 2000
NEC = EV // ECH


def _tc_edge_body(w0_ref, g_ref, Wblk_ref, b_ref, bng_ref, bnb_ref, F_ref,
                  wn_ref, sn_ref, t_all, sum_ref, sq_ref):
    p = pl.program_id(0)
    k = pl.program_id(1)

    @pl.when(p == 0)
    def _pass1():
        @pl.when(k == 0)
        def _init():
            sum_ref[...] = jnp.zeros_like(sum_ref)
            sq_ref[...] = jnp.zeros_like(sq_ref)

        t = (jnp.dot(w0_ref[...], Wblk_ref[...],
                     preferred_element_type=jnp.float32)
             + b_ref[...] + g_ref[...])
        t_all[pl.ds(k * ECH, ECH), :] = t
        sum_ref[...] += jnp.sum(t, axis=0, keepdims=True)
        sq_ref[...] += jnp.sum(t * t, axis=0, keepdims=True)

    @pl.when(p == 1)
    def _pass2():
        inv_n = 1.0 / N_EDGES
        mu = jnp.dot(sum_ref[...], F_ref[...],
                     preferred_element_type=jnp.float32) * inv_n
        ex2 = jnp.dot(sq_ref[...], F_ref[...],
                      preferred_element_type=jnp.float32) * inv_n
        inv = lax.rsqrt(ex2 - mu * mu + 1e-5)
        t = t_all[pl.ds(k * ECH, ECH), :]
        y = bng_ref[...] * (t - mu) * inv + bnb_ref[...]
        o = w0_ref[...] + y * (1.0 / (1.0 + jnp.exp(-y)))
        wn_ref[...] = o
        sn_ref[...] = 1.0 / (1.0 + jnp.exp(-o))


def _tc_edge_call(w0v, gv, Wblk, b128, bng128, bnb128, F):
    row = pl.BlockSpec((ECH, 128), lambda p, k: (k, 0))
    row_p0 = pl.BlockSpec((ECH, 128), lambda p, k: (k * (1 - p), 0))
    row_p1 = pl.BlockSpec((ECH, 128), lambda p, k: (k * p, 0))
    vec = pl.BlockSpec((1, 128), lambda p, k: (0, 0))
    mat = pl.BlockSpec((128, 128), lambda p, k: (0, 0))
    return pl.pallas_call(
        _tc_edge_body,
        grid=(2, NEC),
        in_specs=[row, row_p0, mat, vec, vec, vec, mat],
        out_specs=[row_p1, row_p1],
        out_shape=[jax.ShapeDtypeStruct((EV, 128), jnp.float32),
                   jax.ShapeDtypeStruct((EV, 128), jnp.float32)],
        scratch_shapes=[pltpu.VMEM((EV, 128), jnp.float32),
                        pltpu.VMEM((1, 128), jnp.float32),
                        pltpu.VMEM((1, 128), jnp.float32)],
    )(w0v, gv, Wblk, b128, bng128, bnb128, F)


def kernel(x, edge_index, edge_attr, v_lin0_W, v_lin0_b, v1_W, v1_b, v2_W, v2_b, v3_W, v3_b, v4_W, v4_b, vbn_g, vbn_b, e_lin0_W, e_lin0_b, e0_W, e0_b, ebn_g, ebn_b, se_W1, se_W2):
    src = edge_index[0]
    dst = edge_index[1]
    src3 = src.reshape(NW, NCH, G)
    dst3 = dst.reshape(NW, NCH, G)

    cntp = _sc_cnt_kernel(src)[0]
    cntinv = (1.0 / jnp.maximum(cntp[0, :, 0] + cntp[1, :, 0], 1.0))[:, None]

    h = jax.nn.silu(x @ v_lin0_W.T + v_lin0_b)
    w = jax.nn.silu(edge_attr @ e_lin0_W.T + e_lin0_b)
    s = jax.nn.sigmoid(w)
    wv = w.reshape(EV, 128)

    eye4 = jnp.eye(4, dtype=jnp.float32)
    F = jnp.tile(jnp.eye(32, dtype=jnp.float32), (4, 4))
    Wblk = jnp.einsum("ab,iuv->iaubv", eye4, jnp.transpose(e0_W, (0, 2, 1)))
    Wblk = Wblk.reshape(DEPTH, 128, 128)
    eb128 = jnp.tile(e0_b, (1, 4)).reshape(DEPTH, 1, 128)
    ebg128 = jnp.tile(ebn_g, (1, 4)).reshape(DEPTH, 1, 128)
    ebb128 = jnp.tile(ebn_b, (1, 4)).reshape(DEPTH, 1, 128)

    Wcat = jnp.concatenate(
        [jnp.transpose(v1_W, (0, 2, 1)), jnp.transpose(v2_W, (0, 2, 1)),
         jnp.transpose(v3_W, (0, 2, 1)), jnp.transpose(v4_W, (0, 2, 1))],
        axis=2)
    bcat = jnp.concatenate([v1_b, v2_b, v3_b, v4_b], axis=1)
    bcat = bcat.reshape(DEPTH, 1, 128)
    W1pad = jnp.zeros((UNITS, UNITS), jnp.float32).at[:, :2].set(se_W1.T)
    W2pad = jnp.zeros((UNITS, UNITS), jnp.float32).at[:2, :].set(se_W2.T)
    W1blk = jnp.einsum("ab,uv->aubv", eye4, W1pad).reshape(128, 128)
    W2blk = jnp.einsum("ab,uv->aubv", eye4, W2pad).reshape(128, 128)
    eye128 = jnp.eye(128, dtype=jnp.float32)
    S1, S2, S3, S4 = (eye128[:, q * UNITS:(q + 1) * UNITS] for q in range(4))
    vg = vbn_g.reshape(DEPTH, 1, UNITS)
    vb = vbn_b.reshape(DEPTH, 1, UNITS)

    x1, x2, x3, x4 = _tc_node0_call(h, Wcat[0], bcat[0], W1blk, W2blk,
                                    S1, S2, S3, S4)
    for i in range(DEPTH):
        g, aggp = _sc_edge_kernel(src3, dst3, s, x2, x3, x4)

        wv, sv = _tc_edge_call(wv, g.reshape(EV, 128), Wblk[i], eb128[i],
                               ebg128[i], ebb128[i], F)
        s = sv.reshape(N_EDGES, UNITS)
        if i < DEPTH - 1:
            h, x1, x2, x3, x4 = _tc_node_call(
                h, x1, aggp, cntinv, vg[i], vb[i], Wcat[i + 1], bcat[i + 1],
                W1blk, W2blk, S1, S2, S3, S4)
    return wv.reshape(N_EDGES, UNITS)


# speedup vs baseline: 3.8025x; 1.0424x over previous
import functools

import jax
import jax.numpy as jnp
from jax import lax
from jax.experimental import pallas as pl
from jax.experimental.pallas import tpu as pltpu
from jax.experimental.pallas import tpu_sc as plsc

DEPTH = 12
UNITS = 32
N_NODES = 10000
N_EDGES = 320000

NC = 2
NS = 16
NW = NC * NS
EPW = N_EDGES // NW
G = 80
NCH = EPW // G
NBUF = 5
NRD = 10
ROWS_RD = N_NODES // NRD
RD2 = ROWS_RD // 2
CNT_W = 16


def _sigmoid(v):
    return 1.0 / (1.0 + jnp.exp(-v))


def _sc_edge_body(src3_hbm, dst3_hbm, s_hbm, x2_hbm, x3_hbm, x4_hbm,
                  g_hbm, aggp_hbm,
                  srcA, dstA, wv, r2, r3, r4, cbuf, gbuf, ob,
                  shared_agg, lsem, wsem):
    cid = lax.axis_index("c")
    sid = lax.axis_index("s")
    wid = sid * NC + cid
    base = wid * EPW

    @pl.when(sid < NRD)
    def _zero():
        def _zrow(r, _):
            z = jnp.zeros((16,), jnp.float32)
            ob[r, pl.ds(0, 16)] = z
            ob[r, pl.ds(16, 16)] = z
            return _
        lax.fori_loop(0, RD2, _zrow, None)
        pltpu.sync_copy(ob, shared_agg.at[pl.ds(sid * ROWS_RD, RD2)])
        pltpu.sync_copy(ob, shared_agg.at[pl.ds(sid * ROWS_RD + RD2, RD2)])

    pltpu.sync_copy(src3_hbm.at[wid], srcA)
    pltpu.sync_copy(dst3_hbm.at[wid], dstA)
    plsc.subcore_barrier()

    def _fire(gi, b):
        pltpu.async_copy(x2_hbm.at[dstA.at[gi]], r2.at[b], lsem.at[b])
        pltpu.async_copy(x3_hbm.at[srcA.at[gi]], r3.at[b], lsem.at[b])
        pltpu.async_copy(x4_hbm.at[dstA.at[gi]], r4.at[b], lsem.at[b])
        pltpu.async_copy(s_hbm.at[pl.ds(base + gi * G, G)], wv.at[b], lsem.at[b])

    def _drain_loads(b):
        for _ in range(4):
            pltpu.make_async_copy(
                s_hbm.at[pl.ds(base, G)], wv.at[b], lsem.at[b]).wait()

    def _drain_gwrite(b):
        pltpu.make_async_copy(
            gbuf.at[b], g_hbm.at[pl.ds(base, G)], wsem.at[b]).wait()

    for b in range(NBUF):
        _fire(b, b)

    def _outer(it, _):
        g0 = it * NBUF
        for b in range(NBUF):
            gi = g0 + b
            _drain_loads(b)

            @pl.when(it > 0)
            def _w():
                _drain_gwrite(b)

            def _row(r, _):
                for h in (0, 16):
                    sig = 1.0 / (1.0 + jnp.exp(-wv[b, r, pl.ds(h, 16)]))
                    cbuf[r, pl.ds(h, 16)] = sig * r2[b, r, pl.ds(h, 16)]
                    gbuf[b, r, pl.ds(h, 16)] = (
                        r3[b, r, pl.ds(h, 16)] + r4[b, r, pl.ds(h, 16)])
                return _
            lax.fori_loop(0, G, _row, None)

            @pl.when(gi + NBUF < NCH)
            def _f():
                _fire(gi + NBUF, b)

            pltpu.async_copy(gbuf.at[b], g_hbm.at[pl.ds(base + gi * G, G)],
                             wsem.at[b])
            pltpu.sync_copy(cbuf, shared_agg.at[srcA.at[gi]], add=True)
        return _

    lax.fori_loop(0, NCH // NBUF, _outer, None)
    for b in range(NBUF):
        _drain_gwrite(b)
    plsc.subcore_barrier()

    @pl.when(sid < NRD)
    def _readout():
        for half in range(2):
            pltpu.sync_copy(
                shared_agg.at[pl.ds(sid * ROWS_RD + half * RD2, RD2)], ob)
            pltpu.sync_copy(
                ob, aggp_hbm.at[cid, pl.ds(sid * ROWS_RD + half * RD2, RD2)])


_sc_edge_kernel = functools.partial(
    pl.kernel,
    _sc_edge_body,
    out_type=[
        jax.ShapeDtypeStruct((N_EDGES, UNITS), jnp.float32),
        jax.ShapeDtypeStruct((NC, N_NODES, UNITS), jnp.float32),
    ],
    mesh=plsc.VectorSubcoreMesh(core_axis_name="c", subcore_axis_name="s"),
    compiler_params=pltpu.CompilerParams(use_tc_tiling_on_sc=False),
    scratch_types=[
        pltpu.VMEM((NCH, G), jnp.int32),
        pltpu.VMEM((NCH, G), jnp.int32),
        pltpu.VMEM((NBUF, G, UNITS), jnp.float32),
        pltpu.VMEM((NBUF, G, UNITS), jnp.float32),
        pltpu.VMEM((NBUF, G, UNITS), jnp.float32),
        pltpu.VMEM((NBUF, G, UNITS), jnp.float32),
        pltpu.VMEM((G, UNITS), jnp.float32),
        pltpu.VMEM((NBUF, G, UNITS), jnp.float32),
        pltpu.VMEM((RD2, UNITS), jnp.float32),
        pltpu.VMEM_SHARED((N_NODES, UNITS), jnp.float32),
        pltpu.SemaphoreType.DMA((NBUF,)),
        pltpu.SemaphoreType.DMA((NBUF,)),
    ],
)()


def _sc_cnt_body(src_hbm, cntp_hbm, srcv, ones_buf, ob, shared_cnt):
    cid = lax.axis_index("c")
    sid = lax.axis_index("s")
    wid = sid * NC + cid

    @pl.when(sid < NRD)
    def _zero():
        def _zrow(r, _):
            ob[r, pl.ds(0, 16)] = jnp.zeros((16,), jnp.float32)
            return _
        lax.fori_loop(0, ROWS_RD, _zrow, None)
        pltpu.sync_copy(ob, shared_cnt.at[pl.ds(sid * ROWS_RD, ROWS_RD)])

    def _fill1(r, _):
        ones_buf[r, pl.ds(0, 16)] = jnp.ones((16,), jnp.float32)
        return _
    lax.fori_loop(0, G, _fill1, None)
    plsc.subcore_barrier()

    base = wid * EPW

    def _group(gi, _):
        pltpu.sync_copy(src_hbm.at[pl.ds(base + gi * G, G)], srcv)
        pltpu.sync_copy(ones_buf, shared_cnt.at[srcv], add=True)
        return _

    lax.fori_loop(0, NCH, _group, None)
    plsc.subcore_barrier()

    @pl.when(sid < NRD)
    def _readout():
        pltpu.sync_copy(shared_cnt.at[pl.ds(sid * ROWS_RD, ROWS_RD)], ob)
        pltpu.sync_copy(ob, cntp_hbm.at[cid, pl.ds(sid * ROWS_RD, ROWS_RD)])


_sc_cnt_kernel = functools.partial(
    pl.kernel,
    _sc_cnt_body,
    out_type=[jax.ShapeDtypeStruct((NC, N_NODES, CNT_W), jnp.float32)],
    mesh=plsc.VectorSubcoreMesh(core_axis_name="c", subcore_axis_name="s"),
    compiler_params=pltpu.CompilerParams(use_tc_tiling_on_sc=False),
    scratch_types=[
        pltpu.VMEM((G,), jnp.int32),
        pltpu.VMEM((G, CNT_W), jnp.float32),
        pltpu.VMEM((ROWS_RD, CNT_W), jnp.float32),
        pltpu.VMEM_SHARED((N_NODES, CNT_W), jnp.float32),
    ],
)()


NVB = 2000
NNB = N_NODES // NVB


def _node_mlp(hn, Wcat_ref, bcat_ref, W1blk_ref, W2blk_ref, S_refs,
              x_refs):
    xl = jnp.dot(hn, Wcat_ref[...], preferred_element_type=jnp.float32)
    xl = xl + bcat_ref[...]
    u = jnp.maximum(jnp.dot(xl, W1blk_ref[...],
                            preferred_element_type=jnp.float32), 0.0)
    y = 1.0 / (1.0 + jnp.exp(-jnp.dot(u, W2blk_ref[...],
                                      preferred_element_type=jnp.float32)))
    xall = xl * y
    for S_ref, x_ref in zip(S_refs, x_refs):
        x_ref[...] = jnp.dot(xall, S_ref[...],
                             preferred_element_type=jnp.float32)


def _tc_node_body(x0_ref, x1p_ref, aggp_ref, ci_ref, bng_ref, bnb_ref,
                  Wcat_ref, bcat_ref, W1blk_ref, W2blk_ref,
                  S1_ref, S2_ref, S3_ref, S4_ref,
                  h_ref, x1_ref, x2_ref, x3_ref, x4_ref,
                  z_all, sum_ref, sq_ref):
    p = pl.program_id(0)
    k = pl.program_id(1)

    @pl.when(p == 0)
    def _pass1():
        @pl.when(k == 0)
        def _init():
            sum_ref[...] = jnp.zeros_like(sum_ref)
            sq_ref[...] = jnp.zeros_like(sq_ref)

        z = x1p_ref[...] + (aggp_ref[0] + aggp_ref[1]) * ci_ref[...]
        z_all[pl.ds(k * NVB, NVB), :] = z
        sum_ref[...] += jnp.sum(z, axis=0, keepdims=True)
        sq_ref[...] += jnp.sum(z * z, axis=0, keepdims=True)

    @pl.when(p == 1)
    def _pass2():
        inv_n = 1.0 / N_NODES
        mu = sum_ref[...] * inv_n
        inv = lax.rsqrt(sq_ref[...] * inv_n - mu * mu + 1e-5)
        z = z_all[pl.ds(k * NVB, NVB), :]
        hb = bng_ref[...] * (z - mu) * inv + bnb_ref[...]
        hn = x0_ref[...] + hb * (1.0 / (1.0 + jnp.exp(-hb)))
        h_ref[...] = hn
        _node_mlp(hn, Wcat_ref, bcat_ref, W1blk_ref, W2blk_ref,
                  (S1_ref, S2_ref, S3_ref, S4_ref),
                  (x1_ref, x2_ref, x3_ref, x4_ref))


def _tc_node_call(x0, x1p, aggp, cntinv, bng, bnb, Wcat, bcat,
                  W1blk, W2blk, S1, S2, S3, S4):
    row_p0 = pl.BlockSpec((NVB, UNITS), lambda p, k: (k * (1 - p), 0))
    row_p1 = pl.BlockSpec((NVB, UNITS), lambda p, k: (k * p, 0))
    agg_sp = pl.BlockSpec((NC, NVB, UNITS), lambda p, k: (0, k * (1 - p), 0))
    ci_sp = pl.BlockSpec((NVB, 1), lambda p, k: (k * (1 - p), 0))
    vec = pl.BlockSpec((1, UNITS), lambda p, k: (0, 0))
    w_sp = pl.BlockSpec((UNITS, 128), lambda p, k: (0, 0))
    v128 = pl.BlockSpec((1, 128), lambda p, k: (0, 0))
    m128 = pl.BlockSpec((128, 128), lambda p, k: (0, 0))
    sel = pl.BlockSpec((128, UNITS), lambda p, k: (0, 0))
    out = jax.ShapeDtypeStruct((N_NODES, UNITS), jnp.float32)
    return pl.pallas_call(
        _tc_node_body,
        grid=(2, NNB),
        in_specs=[row_p1, row_p0, agg_sp, ci_sp, vec, vec,
                  w_sp, v128, m128, m128, sel, sel, sel, sel],
        out_specs=[row_p1] * 5,
        out_shape=[out] * 5,
        scratch_shapes=[pltpu.VMEM((N_NODES, UNITS), jnp.float32),
                        pltpu.VMEM((1, UNITS), jnp.float32),
                        pltpu.VMEM((1, UNITS), jnp.float32)],
    )(x0, x1p, aggp, cntinv, bng, bnb, Wcat, bcat, W1blk, W2blk,
      S1, S2, S3, S4)


def _tc_node0_body(h_ref, Wcat_ref, bcat_ref, W1blk_ref, W2blk_ref,
                   S1_ref, S2_ref, S3_ref, S4_ref,
                   x1_ref, x2_ref, x3_ref, x4_ref):
    _node_mlp(h_ref[...], Wcat_ref, bcat_ref, W1blk_ref, W2blk_ref,
              (S1_ref, S2_ref, S3_ref, S4_ref),
              (x1_ref, x2_ref, x3_ref, x4_ref))


def _tc_node0_call(h, Wcat, bcat, W1blk, W2blk, S1, S2, S3, S4):
    row = pl.BlockSpec((NVB, UNITS), lambda k: (k, 0))
    w_sp = pl.BlockSpec((UNITS, 128), lambda k: (0, 0))
    v128 = pl.BlockSpec((1, 128), lambda k: (0, 0))
    m128 = pl.BlockSpec((128, 128), lambda k: (0, 0))
    sel = pl.BlockSpec((128, UNITS), lambda k: (0, 0))
    out = jax.ShapeDtypeStruct((N_NODES, UNITS), jnp.float32)
    return pl.pallas_call(
        _tc_node0_body,
        grid=(NNB,),
        in_specs=[row, w_sp, v128, m128, m128, sel, sel, sel, sel],
        out_specs=[row] * 4,
        out_shape=[out] * 4,
    )(h, Wcat, bcat, W1blk, W2blk, S1, S2, S3, S4)


EV = N_EDGES // 4
ECH = 2000
NEC = EV // ECH


def _tc_edge_body(w0_ref, g_ref, Wblk_ref, b_ref, bng_ref, bnb_ref, F_ref,
                  wn_ref, t_all, sum_ref, sq_ref):
    p = pl.program_id(0)
    k = pl.program_id(1)

    @pl.when(p == 0)
    def _pass1():
        @pl.when(k == 0)
        def _init():
            sum_ref[...] = jnp.zeros_like(sum_ref)
            sq_ref[...] = jnp.zeros_like(sq_ref)

        t = (jnp.dot(w0_ref[...], Wblk_ref[...],
                     preferred_element_type=jnp.float32)
             + b_ref[...] + g_ref[...])
        t_all[pl.ds(k * ECH, ECH), :] = t
        sum_ref[...] += jnp.sum(t, axis=0, keepdims=True)
        sq_ref[...] += jnp.sum(t * t, axis=0, keepdims=True)

    @pl.when(p == 1)
    def _pass2():
        inv_n = 1.0 / N_EDGES
        mu = jnp.dot(sum_ref[...], F_ref[...],
                     preferred_element_type=jnp.float32) * inv_n
        ex2 = jnp.dot(sq_ref[...], F_ref[...],
                      preferred_element_type=jnp.float32) * inv_n
        inv = lax.rsqrt(ex2 - mu * mu + 1e-5)
        t = t_all[pl.ds(k * ECH, ECH), :]
        y = bng_ref[...] * (t - mu) * inv + bnb_ref[...]
        wn_ref[...] = w0_ref[...] + y * (1.0 / (1.0 + jnp.exp(-y)))


def _tc_edge_call(w0v, gv, Wblk, b128, bng128, bnb128, F):
    row = pl.BlockSpec((ECH, 128), lambda p, k: (k, 0))
    row_p0 = pl.BlockSpec((ECH, 128), lambda p, k: (k * (1 - p), 0))
    row_p1 = pl.BlockSpec((ECH, 128), lambda p, k: (k * p, 0))
    vec = pl.BlockSpec((1, 128), lambda p, k: (0, 0))
    mat = pl.BlockSpec((128, 128), lambda p, k: (0, 0))
    return pl.pallas_call(
        _tc_edge_body,
        grid=(2, NEC),
        in_specs=[row, row_p0, mat, vec, vec, vec, mat],
        out_specs=[row_p1],
        out_shape=[jax.ShapeDtypeStruct((EV, 128), jnp.float32)],
        scratch_shapes=[pltpu.VMEM((EV, 128), jnp.float32),
                        pltpu.VMEM((1, 128), jnp.float32),
                        pltpu.VMEM((1, 128), jnp.float32)],
    )(w0v, gv, Wblk, b128, bng128, bnb128, F)


def kernel(x, edge_index, edge_attr, v_lin0_W, v_lin0_b, v1_W, v1_b, v2_W, v2_b, v3_W, v3_b, v4_W, v4_b, vbn_g, vbn_b, e_lin0_W, e_lin0_b, e0_W, e0_b, ebn_g, ebn_b, se_W1, se_W2):
    src = edge_index[0]
    dst = edge_index[1]
    src3 = src.reshape(NW, NCH, G)
    dst3 = dst.reshape(NW, NCH, G)

    cntp = _sc_cnt_kernel(src)[0]
    cntinv = (1.0 / jnp.maximum(cntp[0, :, 0] + cntp[1, :, 0], 1.0))[:, None]

    h = jax.nn.silu(x @ v_lin0_W.T + v_lin0_b)
    w = jax.nn.silu(edge_attr @ e_lin0_W.T + e_lin0_b)
    wv = w.reshape(EV, 128)

    eye4 = jnp.eye(4, dtype=jnp.float32)
    F = jnp.tile(jnp.eye(32, dtype=jnp.float32), (4, 4))
    Wblk = jnp.einsum("ab,iuv->iaubv", eye4, jnp.transpose(e0_W, (0, 2, 1)))
    Wblk = Wblk.reshape(DEPTH, 128, 128)
    eb128 = jnp.tile(e0_b, (1, 4)).reshape(DEPTH, 1, 128)
    ebg128 = jnp.tile(ebn_g, (1, 4)).reshape(DEPTH, 1, 128)
    ebb128 = jnp.tile(ebn_b, (1, 4)).reshape(DEPTH, 1, 128)

    Wcat = jnp.concatenate(
        [jnp.transpose(v1_W, (0, 2, 1)), jnp.transpose(v2_W, (0, 2, 1)),
         jnp.transpose(v3_W, (0, 2, 1)), jnp.transpose(v4_W, (0, 2, 1))],
        axis=2)
    bcat = jnp.concatenate([v1_b, v2_b, v3_b, v4_b], axis=1)
    bcat = bcat.reshape(DEPTH, 1, 128)
    W1pad = jnp.zeros((UNITS, UNITS), jnp.float32).at[:, :2].set(se_W1.T)
    W2pad = jnp.zeros((UNITS, UNITS), jnp.float32).at[:2, :].set(se_W2.T)
    W1blk = jnp.einsum("ab,uv->aubv", eye4, W1pad).reshape(128, 128)
    W2blk = jnp.einsum("ab,uv->aubv", eye4, W2pad).reshape(128, 128)
    eye128 = jnp.eye(128, dtype=jnp.float32)
    S1, S2, S3, S4 = (eye128[:, q * UNITS:(q + 1) * UNITS] for q in range(4))
    vg = vbn_g.reshape(DEPTH, 1, UNITS)
    vb = vbn_b.reshape(DEPTH, 1, UNITS)

    x1, x2, x3, x4 = _tc_node0_call(h, Wcat[0], bcat[0], W1blk, W2blk,
                                    S1, S2, S3, S4)
    for i in range(DEPTH):
        g, aggp = _sc_edge_kernel(src3, dst3, w, x2, x3, x4)

        [wv] = _tc_edge_call(wv, g.reshape(EV, 128), Wblk[i], eb128[i],
                             ebg128[i], ebb128[i], F)
        w = wv.reshape(N_EDGES, UNITS)
        if i < DEPTH - 1:
            h, x1, x2, x3, x4 = _tc_node_call(
                h, x1, aggp, cntinv, vg[i], vb[i], Wcat[i + 1], bcat[i + 1],
                W1blk, W2blk, S1, S2, S3, S4)
    return wv.reshape(N_EDGES, UNITS)
